# baseline (speedup 1.0000x reference)
import functools

import jax
import jax.numpy as jnp
from jax import lax
from jax.experimental import pallas as pl
from jax.experimental.pallas import tpu as pltpu
from jax.experimental.pallas import tpu_sc as plsc

N = 10000
E = 320000
D_IN = 128
D_HID = 128
D_OUT = 64
DH = 64

NC = 2
NS = 16
NW = NC * NS

BATCH = 128
NB = 80
EP = NB * BATCH
E_PAD = EP * NW
NBT = E_PAD // BATCH
NB_F = NBT // NS
N_PAD = 10240
RPT = N_PAD // NS
SRT = N // NS


def _deg_kernel():
    mesh = plsc.VectorSubcoreMesh(core_axis_name="c", subcore_axis_name="s")

    @functools.partial(
        pl.kernel,
        out_type=jax.ShapeDtypeStruct((NC, N_PAD, 16), jnp.float32),
        mesh=mesh,
        scratch_types=[
            pltpu.VMEM((2, BATCH), jnp.int32),
            pltpu.VMEM((BATCH, 16), jnp.float32),
            pltpu.VMEM((BATCH, 16), jnp.float32),
            pltpu.VMEM_SHARED((N_PAD, 16), jnp.float32),
        ],
        compiler_params=pltpu.CompilerParams(use_tc_tiling_on_sc=False),
    )
    def deg(idx_hbm, ones_hbm, out_hbm, idx_v, ones_v, z_v, acc_sh):
        c = lax.axis_index("c")
        s = lax.axis_index("s")
        wid = s * NC + c
        ibase = wid * NB
        pltpu.sync_copy(ones_hbm, ones_v)
        def zrow(i, _):
            z_v[i, :] = jnp.zeros((16,), jnp.float32)
            return 0
        lax.fori_loop(0, BATCH, zrow, 0)
        for r in range(RPT // BATCH):
            pltpu.sync_copy(z_v, acc_sh.at[pl.ds(s * RPT + r * BATCH, BATCH)])
        plsc.subcore_barrier()

        def body(i, _):
            pltpu.sync_copy(idx_hbm.at[ibase + i], idx_v)
            pltpu.sync_copy(ones_v, acc_sh.at[idx_v.at[1]], add=True)
            return 0

        lax.fori_loop(0, NB, body, 0)
        plsc.subcore_barrier()
        pltpu.sync_copy(acc_sh.at[pl.ds(s * RPT, RPT)],
                        out_hbm.at[c, pl.ds(s * RPT, RPT)])

    return deg


def _zero_init(rows0_v, acc_sh, s):
    def zrow(i, _):
        for j in range(DH // 16):
            rows0_v[i, pl.ds(j * 16, 16)] = jnp.zeros((16,), jnp.float32)
        return 0
    lax.fori_loop(0, BATCH, zrow, 0)
    for r in range(RPT // BATCH):
        pltpu.sync_copy(rows0_v, acc_sh.at[pl.ds(s * RPT + r * BATCH, BATCH)])


def _edge_loop(idx_hbm, table_sh, acc_sh, bufs, ibase, nb):
    for b in (0, 1):
        idx_v, rows_v, sem = bufs[b]
        pltpu.sync_copy(idx_hbm.at[ibase + b], idx_v)
        pltpu.async_copy(table_sh.at[idx_v.at[0]], rows_v, sem)

    def body(k, _):
        for b in (0, 1):
            i = 2 * k + b
            idx_v, rows_v, sem = bufs[b]
            pltpu.make_async_copy(
                table_sh.at[idx_v.at[0]], rows_v, sem).wait()
            pltpu.sync_copy(rows_v, acc_sh.at[idx_v.at[1]], add=True)

            @pl.when(i + 2 < nb)
            def _():
                pltpu.sync_copy(idx_hbm.at[ibase + i + 2], idx_v)
                pltpu.async_copy(table_sh.at[idx_v.at[0]], rows_v, sem)
        return 0

    lax.fori_loop(0, nb // 2, body, 0)


_AGG_SCRATCH = [
    pltpu.VMEM((2, BATCH), jnp.int32),
    pltpu.VMEM((2, BATCH), jnp.int32),
    pltpu.VMEM((BATCH, DH), jnp.float32),
    pltpu.VMEM((BATCH, DH), jnp.float32),
    pltpu.VMEM_SHARED((N_PAD, DH), jnp.float32),
    pltpu.VMEM_SHARED((N_PAD, DH), jnp.float32),
    pltpu.SemaphoreType.DMA,
    pltpu.SemaphoreType.DMA,
]


def _agg_fsplit_kernel():
    mesh = plsc.VectorSubcoreMesh(core_axis_name="c", subcore_axis_name="s")

    @functools.partial(
        pl.kernel,
        out_type=jax.ShapeDtypeStruct((NC, N_PAD, DH), jnp.float32),
        mesh=mesh,
        scratch_types=_AGG_SCRATCH,
        compiler_params=pltpu.CompilerParams(use_tc_tiling_on_sc=False),
    )
    def agg(table_hbm, idx_hbm, out_hbm,
            idx0_v, idx1_v, rows0_v, rows1_v, table_sh, acc_sh, sem0, sem1):
        c = lax.axis_index("c")
        s = lax.axis_index("s")
        bufs = ((idx0_v, rows0_v, sem0), (idx1_v, rows1_v, sem1))
        pltpu.sync_copy(table_hbm.at[c, pl.ds(s * SRT, SRT)],
                        table_sh.at[pl.ds(s * SRT, SRT)])
        _zero_init(rows0_v, acc_sh, s)
        plsc.subcore_barrier()
        _edge_loop(idx_hbm, table_sh, acc_sh, bufs, s * NB_F, NB_F)
        plsc.subcore_barrier()
        pltpu.sync_copy(acc_sh.at[pl.ds(s * RPT, RPT)],
                        out_hbm.at[c, pl.ds(s * RPT, RPT)])

    return agg


def _agg_esplit_kernel():
    mesh = plsc.VectorSubcoreMesh(core_axis_name="c", subcore_axis_name="s")

    @functools.partial(
        pl.kernel,
        out_type=jax.ShapeDtypeStruct((NC, N_PAD, DH), jnp.float32),
        mesh=mesh,
        scratch_types=_AGG_SCRATCH,
        compiler_params=pltpu.CompilerParams(use_tc_tiling_on_sc=False),
    )
    def agg(table_hbm, idx_hbm, out_hbm,
            idx0_v, idx1_v, rows0_v, rows1_v, table_sh, acc_sh, sem0, sem1):
        c = lax.axis_index("c")
        s = lax.axis_index("s")
        wid = s * NC + c
        bufs = ((idx0_v, rows0_v, sem0), (idx1_v, rows1_v, sem1))
        pltpu.sync_copy(table_hbm.at[pl.ds(s * SRT, SRT)],
                        table_sh.at[pl.ds(s * SRT, SRT)])
        _zero_init(rows0_v, acc_sh, s)
        plsc.subcore_barrier()
        _edge_loop(idx_hbm, table_sh, acc_sh, bufs, wid * NB, NB)
        plsc.subcore_barrier()
        pltpu.sync_copy(acc_sh.at[pl.ds(s * RPT, RPT)],
                        out_hbm.at[c, pl.ds(s * RPT, RPT)])

    return agg


_ROWS_BLK = 1000
_GRID = N // _ROWS_BLK


def _dis_from(degp_blk):
    deg = degp_blk[0, :, :1] + degp_blk[1, :, :1] + 1.0
    return lax.rsqrt(deg)


def _tc1_body(degp_ref, x_ref, w1_ref, g1_ref):
    dis = _dis_from(degp_ref[...])
    h = jnp.dot(x_ref[...], w1_ref[...], preferred_element_type=jnp.float32)
    g1_ref[0] = dis * h[:, :DH]
    g1_ref[1] = dis * h[:, DH:]


def _tc2_body(degp_ref, r1_ref, g1_ref, b1_ref, w2_ref, g2_ref):
    dis = _dis_from(degp_ref[...])
    a_lo = dis * (r1_ref[0] + g1_ref[0]) + b1_ref[:, :DH]
    a_hi = dis * (r1_ref[1] + g1_ref[1]) + b1_ref[:, DH:]
    x2 = jnp.concatenate([jnp.maximum(a_lo, 0.0), jnp.maximum(a_hi, 0.0)],
                         axis=1)
    g2_ref[...] = dis * jnp.dot(x2, w2_ref[...],
                                preferred_element_type=jnp.float32)


def _tc3_body(degp_ref, r2_ref, g2_ref, b2_ref, out_ref):
    dis = _dis_from(degp_ref[...])
    out_ref[...] = dis * (r2_ref[0] + r2_ref[1] + g2_ref[...]) + b2_ref[...]


def _blk_parts(d):
    return pl.BlockSpec((NC, _ROWS_BLK, d), lambda i: (0, i, 0))


def _blk_rows(d):
    return pl.BlockSpec((_ROWS_BLK, d), lambda i: (i, 0))


def _blk_full(shape):
    return pl.BlockSpec(shape, lambda i: tuple(0 for _ in shape))


def kernel(x, edge_index, W1, b1, W2, b2):
    src = edge_index[0]
    dst = edge_index[1]
    pad = E_PAD - E
    src_p = jnp.concatenate([src, jnp.zeros((pad,), jnp.int32)])
    dst_p = jnp.concatenate([dst, jnp.full((pad,), N, jnp.int32)])
    idx = (jnp.stack([src_p, dst_p])
           .reshape(2, NBT, BATCH)
           .transpose(1, 0, 2))
    ones16 = jnp.ones((BATCH, 16), jnp.float32)

    degp = _deg_kernel()(idx, ones16)

    g1 = pl.pallas_call(
        _tc1_body,
        grid=(_GRID,),
        in_specs=[_blk_parts(16), _blk_rows(D_IN), _blk_full((D_IN, D_HID))],
        out_specs=_blk_parts(DH),
        out_shape=jax.ShapeDtypeStruct((NC, N, DH), jnp.float32),
    )(degp, x, W1)

    r1 = _agg_fsplit_kernel()(g1, idx)

    g2 = pl.pallas_call(
        _tc2_body,
        grid=(_GRID,),
        in_specs=[_blk_parts(16), _blk_parts(DH), _blk_parts(DH),
                  _blk_full((1, D_HID)), _blk_full((D_HID, D_OUT))],
        out_specs=_blk_rows(D_OUT),
        out_shape=jax.ShapeDtypeStruct((N, D_OUT), jnp.float32),
    )(degp, r1, g1, b1.reshape(1, D_HID), W2)

    r2 = _agg_esplit_kernel()(g2, idx)

    out = pl.pallas_call(
        _tc3_body,
        grid=(_GRID,),
        in_specs=[_blk_parts(16), _blk_parts(D_OUT), _blk_rows(D_OUT),
                  _blk_full((1, D_OUT))],
        out_specs=_blk_rows(D_OUT),
        out_shape=jax.ShapeDtypeStruct((N, D_OUT), jnp.float32),
    )(degp, r2, g2, b2.reshape(1, D_OUT))

    return out


# speedup vs baseline: 1.0719x; 1.0719x over previous
import functools

import jax
import jax.numpy as jnp
from jax import lax
from jax.experimental import pallas as pl
from jax.experimental.pallas import tpu as pltpu
from jax.experimental.pallas import tpu_sc as plsc

N = 10000
E = 320000
D_IN = 128
D_HID = 128
D_OUT = 64
DH = 64

NC = 2
NS = 16
NW = NC * NS

BATCH = 128
NBC = 2
EPT = NBC * BATCH
NB = 80
EP = NB * BATCH
E_PAD = EP * NW
NBT = E_PAD // BATCH
NB_F = NBT // NS
N_PAD = 10240
RPT = N_PAD // NS
SRT = N // NS


def _deg_kernel():
    mesh = plsc.VectorSubcoreMesh(core_axis_name="c", subcore_axis_name="s")

    NBC_D = 4

    @functools.partial(
        pl.kernel,
        out_type=jax.ShapeDtypeStruct((NC, N_PAD, 16), jnp.float32),
        mesh=mesh,
        scratch_types=[
            pltpu.VMEM((2, NBC_D * BATCH), jnp.int32),
            pltpu.VMEM((NBC_D * BATCH, 16), jnp.float32),
            pltpu.VMEM_SHARED((N_PAD, 16), jnp.float32),
        ],
        compiler_params=pltpu.CompilerParams(use_tc_tiling_on_sc=False),
    )
    def deg(idx_hbm, ones_hbm, out_hbm, idx_v, ones_v, acc_sh):
        c = lax.axis_index("c")
        s = lax.axis_index("s")
        wid = s * NC + c
        ibase = wid * NB
        def zrow(i, _):
            ones_v[i, :] = jnp.zeros((16,), jnp.float32)
            return 0
        lax.fori_loop(0, BATCH, zrow, 0)
        for r in range(RPT // BATCH):
            pltpu.sync_copy(ones_v.at[pl.ds(0, BATCH)],
                            acc_sh.at[pl.ds(s * RPT + r * BATCH, BATCH)])
        pltpu.sync_copy(ones_hbm, ones_v)
        plsc.subcore_barrier()

        def body(i, _):
            pltpu.sync_copy(
                idx_hbm.at[1, pl.ds((ibase + i * NBC_D) * BATCH,
                                    NBC_D * BATCH)],
                idx_v.at[1])
            pltpu.sync_copy(ones_v, acc_sh.at[idx_v.at[1]], add=True)
            return 0

        lax.fori_loop(0, NB // NBC_D, body, 0)
        plsc.subcore_barrier()
        pltpu.sync_copy(acc_sh.at[pl.ds(s * RPT, RPT)],
                        out_hbm.at[c, pl.ds(s * RPT, RPT)])

    return deg


def _zero_init(rows0_v, acc_sh, s):
    def zrow(i, _):
        for j in range(DH // 16):
            rows0_v[i, pl.ds(j * 16, 16)] = jnp.zeros((16,), jnp.float32)
        return 0
    lax.fori_loop(0, BATCH, zrow, 0)
    for r in range(RPT // BATCH):
        pltpu.sync_copy(rows0_v.at[pl.ds(0, BATCH)],
                        acc_sh.at[pl.ds(s * RPT + r * BATCH, BATCH)])


def _copy_idx(idx_hbm, idx_v, mb):
    for p in (0, 1):
        pltpu.sync_copy(idx_hbm.at[p, pl.ds(mb * EPT, EPT)], idx_v.at[p])


def _edge_loop(idx_hbm, table_sh, acc_sh, bufs, mb_base, nmb):
    for b in (0, 1):
        idx_v, rows_v, sem = bufs[b]
        _copy_idx(idx_hbm, idx_v, mb_base + b)
        pltpu.async_copy(table_sh.at[idx_v.at[0]], rows_v, sem)

    def body(k, _):
        for b in (0, 1):
            i = 2 * k + b
            idx_v, rows_v, sem = bufs[b]
            pltpu.make_async_copy(
                table_sh.at[idx_v.at[0]], rows_v, sem).wait()
            pltpu.sync_copy(rows_v, acc_sh.at[idx_v.at[1]], add=True)

            @pl.when(i + 2 < nmb)
            def _():
                _copy_idx(idx_hbm, idx_v, mb_base + i + 2)
                pltpu.async_copy(table_sh.at[idx_v.at[0]], rows_v, sem)
        return 0

    lax.fori_loop(0, nmb // 2, body, 0)


_AGG_SCRATCH = [
    pltpu.VMEM((2, EPT), jnp.int32),
    pltpu.VMEM((2, EPT), jnp.int32),
    pltpu.VMEM((EPT, DH), jnp.float32),
    pltpu.VMEM((EPT, DH), jnp.float32),
    pltpu.VMEM_SHARED((N_PAD, DH), jnp.float32),
    pltpu.VMEM_SHARED((N_PAD, DH), jnp.float32),
    pltpu.SemaphoreType.DMA,
    pltpu.SemaphoreType.DMA,
]


def _agg_fsplit_kernel():
    mesh = plsc.VectorSubcoreMesh(core_axis_name="c", subcore_axis_name="s")

    @functools.partial(
        pl.kernel,
        out_type=jax.ShapeDtypeStruct((NC, N_PAD, DH), jnp.float32),
        mesh=mesh,
        scratch_types=_AGG_SCRATCH,
        compiler_params=pltpu.CompilerParams(use_tc_tiling_on_sc=False),
    )
    def agg(table_hbm, idx_hbm, out_hbm,
            idx0_v, idx1_v, rows0_v, rows1_v, table_sh, acc_sh, sem0, sem1):
        c = lax.axis_index("c")
        s = lax.axis_index("s")
        bufs = ((idx0_v, rows0_v, sem0), (idx1_v, rows1_v, sem1))
        pltpu.sync_copy(table_hbm.at[c, pl.ds(s * SRT, SRT)],
                        table_sh.at[pl.ds(s * SRT, SRT)])
        _zero_init(rows0_v, acc_sh, s)
        plsc.subcore_barrier()
        _edge_loop(idx_hbm, table_sh, acc_sh, bufs,
                   s * (NB_F // NBC), NB_F // NBC)
        plsc.subcore_barrier()
        pltpu.sync_copy(acc_sh.at[pl.ds(s * RPT, RPT)],
                        out_hbm.at[c, pl.ds(s * RPT, RPT)])

    return agg


def _agg_esplit_kernel():
    mesh = plsc.VectorSubcoreMesh(core_axis_name="c", subcore_axis_name="s")

    @functools.partial(
        pl.kernel,
        out_type=jax.ShapeDtypeStruct((NC, N_PAD, DH), jnp.float32),
        mesh=mesh,
        scratch_types=_AGG_SCRATCH,
        compiler_params=pltpu.CompilerParams(use_tc_tiling_on_sc=False),
    )
    def agg(table_hbm, idx_hbm, out_hbm,
            idx0_v, idx1_v, rows0_v, rows1_v, table_sh, acc_sh, sem0, sem1):
        c = lax.axis_index("c")
        s = lax.axis_index("s")
        wid = s * NC + c
        bufs = ((idx0_v, rows0_v, sem0), (idx1_v, rows1_v, sem1))
        pltpu.sync_copy(table_hbm.at[pl.ds(s * SRT, SRT)],
                        table_sh.at[pl.ds(s * SRT, SRT)])
        _zero_init(rows0_v, acc_sh, s)
        plsc.subcore_barrier()
        _edge_loop(idx_hbm, table_sh, acc_sh, bufs,
                   wid * (NB // NBC), NB // NBC)
        plsc.subcore_barrier()
        pltpu.sync_copy(acc_sh.at[pl.ds(s * RPT, RPT)],
                        out_hbm.at[c, pl.ds(s * RPT, RPT)])

    return agg


_ROWS_BLK = 1000
_GRID = N // _ROWS_BLK


def _dis_from(degp_blk):
    deg = degp_blk[0, :, :1] + degp_blk[1, :, :1] + 1.0
    return lax.rsqrt(deg)


def _tc1_body(degp_ref, x_ref, w1_ref, g1_ref):
    dis = _dis_from(degp_ref[...])
    h = jnp.dot(x_ref[...], w1_ref[...], preferred_element_type=jnp.float32)
    g1_ref[0] = dis * h[:, :DH]
    g1_ref[1] = dis * h[:, DH:]


def _tc2_body(degp_ref, r1_ref, g1_ref, b1_ref, w2_ref, g2_ref):
    dis = _dis_from(degp_ref[...])
    a_lo = dis * (r1_ref[0] + g1_ref[0]) + b1_ref[:, :DH]
    a_hi = dis * (r1_ref[1] + g1_ref[1]) + b1_ref[:, DH:]
    x2 = jnp.concatenate([jnp.maximum(a_lo, 0.0), jnp.maximum(a_hi, 0.0)],
                         axis=1)
    g2_ref[...] = dis * jnp.dot(x2, w2_ref[...],
                                preferred_element_type=jnp.float32)


def _tc3_body(degp_ref, r2_ref, g2_ref, b2_ref, out_ref):
    dis = _dis_from(degp_ref[...])
    out_ref[...] = dis * (r2_ref[0] + r2_ref[1] + g2_ref[...]) + b2_ref[...]


def _blk_parts(d):
    return pl.BlockSpec((NC, _ROWS_BLK, d), lambda i: (0, i, 0))


def _blk_rows(d):
    return pl.BlockSpec((_ROWS_BLK, d), lambda i: (i, 0))


def _blk_full(shape):
    return pl.BlockSpec(shape, lambda i: tuple(0 for _ in shape))


def kernel(x, edge_index, W1, b1, W2, b2):
    src = edge_index[0]
    dst = edge_index[1]
    pad = E_PAD - E
    src_p = jnp.concatenate([src, jnp.zeros((pad,), jnp.int32)])
    dst_p = jnp.concatenate([dst, jnp.full((pad,), N, jnp.int32)])
    idx = jnp.stack([src_p, dst_p])
    ones16 = jnp.ones((4 * BATCH, 16), jnp.float32)

    degp = _deg_kernel()(idx, ones16)

    g1 = pl.pallas_call(
        _tc1_body,
        grid=(_GRID,),
        in_specs=[_blk_parts(16), _blk_rows(D_IN), _blk_full((D_IN, D_HID))],
        out_specs=_blk_parts(DH),
        out_shape=jax.ShapeDtypeStruct((NC, N, DH), jnp.float32),
    )(degp, x, W1)

    r1 = _agg_fsplit_kernel()(g1, idx)

    g2 = pl.pallas_call(
        _tc2_body,
        grid=(_GRID,),
        in_specs=[_blk_parts(16), _blk_parts(DH), _blk_parts(DH),
                  _blk_full((1, D_HID)), _blk_full((D_HID, D_OUT))],
        out_specs=_blk_rows(D_OUT),
        out_shape=jax.ShapeDtypeStruct((N, D_OUT), jnp.float32),
    )(degp, r1, g1, b1.reshape(1, D_HID), W2)

    r2 = _agg_esplit_kernel()(g2, idx)

    out = pl.pallas_call(
        _tc3_body,
        grid=(_GRID,),
        in_specs=[_blk_parts(16), _blk_parts(D_OUT), _blk_rows(D_OUT),
                  _blk_full((1, D_OUT))],
        out_specs=_blk_rows(D_OUT),
        out_shape=jax.ShapeDtypeStruct((N, D_OUT), jnp.float32),
    )(degp, r2, g2, b2.reshape(1, D_OUT))

    return out


# speedup vs baseline: 1.1045x; 1.0304x over previous
import functools

import jax
import jax.numpy as jnp
from jax import lax
from jax.experimental import pallas as pl
from jax.experimental.pallas import tpu as pltpu
from jax.experimental.pallas import tpu_sc as plsc

N = 10000
E = 320000
D_IN = 128
D_HID = 128
D_OUT = 64
DH = 64

NC = 2
NS = 16
NW = NC * NS

BATCH = 128
EPT = 320
NB = 80
EP = NB * BATCH
E_PAD = EP * NW
NBT = E_PAD // BATCH
NB_F = NBT // NS
N_PAD = 10240
RPT = N_PAD // NS
SRT = N // NS


def _deg_kernel():
    mesh = plsc.VectorSubcoreMesh(core_axis_name="c", subcore_axis_name="s")

    NBC_D = 8

    @functools.partial(
        pl.kernel,
        out_type=jax.ShapeDtypeStruct((NC, N_PAD, 16), jnp.float32),
        mesh=mesh,
        scratch_types=[
            pltpu.VMEM((2, NBC_D * BATCH), jnp.int32),
            pltpu.VMEM((NBC_D * BATCH, 16), jnp.float32),
            pltpu.VMEM_SHARED((N_PAD, 16), jnp.float32),
        ],
        compiler_params=pltpu.CompilerParams(use_tc_tiling_on_sc=False),
    )
    def deg(idx_hbm, ones_hbm, out_hbm, idx_v, ones_v, acc_sh):
        c = lax.axis_index("c")
        s = lax.axis_index("s")
        wid = s * NC + c
        ibase = wid * NB
        def zrow(i, _):
            ones_v[i, :] = jnp.zeros((16,), jnp.float32)
            return 0
        lax.fori_loop(0, BATCH, zrow, 0)
        for r in range(RPT // BATCH):
            pltpu.sync_copy(ones_v.at[pl.ds(0, BATCH)],
                            acc_sh.at[pl.ds(s * RPT + r * BATCH, BATCH)])
        pltpu.sync_copy(ones_hbm, ones_v)
        plsc.subcore_barrier()

        def body(i, _):
            pltpu.sync_copy(
                idx_hbm.at[1, pl.ds((ibase + i * NBC_D) * BATCH,
                                    NBC_D * BATCH)],
                idx_v.at[1])
            pltpu.sync_copy(ones_v, acc_sh.at[idx_v.at[1]], add=True)
            return 0

        lax.fori_loop(0, NB // NBC_D, body, 0)
        plsc.subcore_barrier()
        pltpu.sync_copy(acc_sh.at[pl.ds(s * RPT, RPT)],
                        out_hbm.at[c, pl.ds(s * RPT, RPT)])

    return deg


def _zero_init(rows0_v, acc_sh, s):
    def zrow(i, _):
        for j in range(DH // 16):
            rows0_v[i, pl.ds(j * 16, 16)] = jnp.zeros((16,), jnp.float32)
        return 0
    lax.fori_loop(0, BATCH, zrow, 0)
    for r in range(RPT // BATCH):
        pltpu.sync_copy(rows0_v.at[pl.ds(0, BATCH)],
                        acc_sh.at[pl.ds(s * RPT + r * BATCH, BATCH)])


def _copy_idx(idx_hbm, idx_v, mb):
    for p in (0, 1):
        pltpu.sync_copy(idx_hbm.at[p, pl.ds(mb * EPT, EPT)], idx_v.at[p])


def _edge_loop(idx_hbm, table_sh, acc_sh, bufs, mb_base, nmb):
    for b in (0, 1):
        idx_v, rows_v, sem = bufs[b]
        _copy_idx(idx_hbm, idx_v, mb_base + b)
        pltpu.async_copy(table_sh.at[idx_v.at[0]], rows_v, sem)

    def body(k, _):
        for b in (0, 1):
            i = 2 * k + b
            idx_v, rows_v, sem = bufs[b]
            pltpu.make_async_copy(
                table_sh.at[idx_v.at[0]], rows_v, sem).wait()
            pltpu.sync_copy(rows_v, acc_sh.at[idx_v.at[1]], add=True)

            @pl.when(i + 2 < nmb)
            def _():
                _copy_idx(idx_hbm, idx_v, mb_base + i + 2)
                pltpu.async_copy(table_sh.at[idx_v.at[0]], rows_v, sem)
        return 0

    lax.fori_loop(0, nmb // 2, body, 0)


_AGG_SCRATCH = [
    pltpu.VMEM((2, EPT), jnp.int32),
    pltpu.VMEM((2, EPT), jnp.int32),
    pltpu.VMEM((EPT, DH), jnp.float32),
    pltpu.VMEM((EPT, DH), jnp.float32),
    pltpu.VMEM_SHARED((N_PAD, DH), jnp.float32),
    pltpu.VMEM_SHARED((N_PAD, DH), jnp.float32),
    pltpu.SemaphoreType.DMA,
    pltpu.SemaphoreType.DMA,
]


def _agg_fsplit_kernel():
    mesh = plsc.VectorSubcoreMesh(core_axis_name="c", subcore_axis_name="s")

    @functools.partial(
        pl.kernel,
        out_type=jax.ShapeDtypeStruct((NC, N_PAD, DH), jnp.float32),
        mesh=mesh,
        scratch_types=_AGG_SCRATCH,
        compiler_params=pltpu.CompilerParams(use_tc_tiling_on_sc=False),
    )
    def agg(table_hbm, idx_hbm, out_hbm,
            idx0_v, idx1_v, rows0_v, rows1_v, table_sh, acc_sh, sem0, sem1):
        c = lax.axis_index("c")
        s = lax.axis_index("s")
        bufs = ((idx0_v, rows0_v, sem0), (idx1_v, rows1_v, sem1))
        pltpu.sync_copy(table_hbm.at[c, pl.ds(s * SRT, SRT)],
                        table_sh.at[pl.ds(s * SRT, SRT)])
        _zero_init(rows0_v, acc_sh, s)
        plsc.subcore_barrier()
        _edge_loop(idx_hbm, table_sh, acc_sh, bufs,
                   s * (E_PAD // NS // EPT), E_PAD // NS // EPT)
        plsc.subcore_barrier()
        pltpu.sync_copy(acc_sh.at[pl.ds(s * RPT, RPT)],
                        out_hbm.at[c, pl.ds(s * RPT, RPT)])

    return agg


def _agg_esplit_kernel():
    mesh = plsc.VectorSubcoreMesh(core_axis_name="c", subcore_axis_name="s")

    @functools.partial(
        pl.kernel,
        out_type=jax.ShapeDtypeStruct((NC, N_PAD, DH), jnp.float32),
        mesh=mesh,
        scratch_types=_AGG_SCRATCH,
        compiler_params=pltpu.CompilerParams(use_tc_tiling_on_sc=False),
    )
    def agg(table_hbm, idx_hbm, out_hbm,
            idx0_v, idx1_v, rows0_v, rows1_v, table_sh, acc_sh, sem0, sem1):
        c = lax.axis_index("c")
        s = lax.axis_index("s")
        wid = s * NC + c
        bufs = ((idx0_v, rows0_v, sem0), (idx1_v, rows1_v, sem1))
        pltpu.sync_copy(table_hbm.at[pl.ds(s * SRT, SRT)],
                        table_sh.at[pl.ds(s * SRT, SRT)])
        _zero_init(rows0_v, acc_sh, s)
        plsc.subcore_barrier()
        _edge_loop(idx_hbm, table_sh, acc_sh, bufs,
                   wid * (EP // EPT), EP // EPT)
        plsc.subcore_barrier()
        pltpu.sync_copy(acc_sh.at[pl.ds(s * RPT, RPT)],
                        out_hbm.at[c, pl.ds(s * RPT, RPT)])

    return agg


_ROWS_BLK = 1000
_GRID = N // _ROWS_BLK


def _dis_from(degp_blk):
    deg = degp_blk[0, :, :1] + degp_blk[1, :, :1] + 1.0
    return lax.rsqrt(deg)


def _tc1_body(degp_ref, x_ref, w1_ref, g1_ref):
    dis = _dis_from(degp_ref[...])
    h = jnp.dot(x_ref[...], w1_ref[...], preferred_element_type=jnp.float32)
    g1_ref[0] = dis * h[:, :DH]
    g1_ref[1] = dis * h[:, DH:]


def _tc2_body(degp_ref, r1_ref, g1_ref, b1_ref, w2_ref, g2_ref):
    dis = _dis_from(degp_ref[...])
    a_lo = dis * (r1_ref[0] + g1_ref[0]) + b1_ref[:, :DH]
    a_hi = dis * (r1_ref[1] + g1_ref[1]) + b1_ref[:, DH:]
    x2 = jnp.concatenate([jnp.maximum(a_lo, 0.0), jnp.maximum(a_hi, 0.0)],
                         axis=1)
    g2_ref[...] = dis * jnp.dot(x2, w2_ref[...],
                                preferred_element_type=jnp.float32)


def _tc3_body(degp_ref, r2_ref, g2_ref, b2_ref, out_ref):
    dis = _dis_from(degp_ref[...])
    out_ref[...] = dis * (r2_ref[0] + r2_ref[1] + g2_ref[...]) + b2_ref[...]


def _blk_parts(d):
    return pl.BlockSpec((NC, _ROWS_BLK, d), lambda i: (0, i, 0))


def _blk_rows(d):
    return pl.BlockSpec((_ROWS_BLK, d), lambda i: (i, 0))


def _blk_full(shape):
    return pl.BlockSpec(shape, lambda i: tuple(0 for _ in shape))


def kernel(x, edge_index, W1, b1, W2, b2):
    src = edge_index[0]
    dst = edge_index[1]
    pad = E_PAD - E
    src_p = jnp.concatenate([src, jnp.zeros((pad,), jnp.int32)])
    dst_p = jnp.concatenate([dst, jnp.full((pad,), N, jnp.int32)])
    idx = jnp.stack([src_p, dst_p])
    ones16 = jnp.ones((8 * BATCH, 16), jnp.float32)

    degp = _deg_kernel()(idx, ones16)

    g1 = pl.pallas_call(
        _tc1_body,
        grid=(_GRID,),
        in_specs=[_blk_parts(16), _blk_rows(D_IN), _blk_full((D_IN, D_HID))],
        out_specs=_blk_parts(DH),
        out_shape=jax.ShapeDtypeStruct((NC, N, DH), jnp.float32),
    )(degp, x, W1)

    r1 = _agg_fsplit_kernel()(g1, idx)

    g2 = pl.pallas_call(
        _tc2_body,
        grid=(_GRID,),
        in_specs=[_blk_parts(16), _blk_parts(DH), _blk_parts(DH),
                  _blk_full((1, D_HID)), _blk_full((D_HID, D_OUT))],
        out_specs=_blk_rows(D_OUT),
        out_shape=jax.ShapeDtypeStruct((N, D_OUT), jnp.float32),
    )(degp, r1, g1, b1.reshape(1, D_HID), W2)

    r2 = _agg_esplit_kernel()(g2, idx)

    out = pl.pallas_call(
        _tc3_body,
        grid=(_GRID,),
        in_specs=[_blk_parts(16), _blk_parts(D_OUT), _blk_rows(D_OUT),
                  _blk_full((1, D_OUT))],
        out_specs=_blk_rows(D_OUT),
        out_shape=jax.ShapeDtypeStruct((N, D_OUT), jnp.float32),
    )(degp, r2, g2, b2.reshape(1, D_OUT))

    return out


# speedup vs baseline: 1.1237x; 1.0174x over previous
import functools

import jax
import jax.numpy as jnp
from jax import lax
from jax.experimental import pallas as pl
from jax.experimental.pallas import tpu as pltpu
from jax.experimental.pallas import tpu_sc as plsc

N = 10000
E = 320000
D_IN = 128
D_HID = 128
D_OUT = 64
DH = 64

NC = 2
NS = 16
NW = NC * NS

BATCH = 128
EPT = 320
NB = 80
EP = NB * BATCH
E_PAD = EP * NW
NBT = E_PAD // BATCH
NB_F = NBT // NS
N_PAD = 10240
RPT = N_PAD // NS
SRT = N // NS


def _deg_kernel():
    mesh = plsc.VectorSubcoreMesh(core_axis_name="c", subcore_axis_name="s")

    NBC_D = 8

    @functools.partial(
        pl.kernel,
        out_type=jax.ShapeDtypeStruct((NC, N_PAD, 16), jnp.float32),
        mesh=mesh,
        scratch_types=[
            pltpu.VMEM((2, NBC_D * BATCH), jnp.int32),
            pltpu.VMEM((NBC_D * BATCH, 16), jnp.float32),
            pltpu.VMEM_SHARED((N_PAD, 16), jnp.float32),
        ],
        compiler_params=pltpu.CompilerParams(use_tc_tiling_on_sc=False),
    )
    def deg(idx_hbm, ones_hbm, out_hbm, idx_v, ones_v, acc_sh):
        c = lax.axis_index("c")
        s = lax.axis_index("s")
        wid = s * NC + c
        ibase = wid * NB
        def zrow(i, _):
            ones_v[i, :] = jnp.zeros((16,), jnp.float32)
            return 0
        lax.fori_loop(0, BATCH, zrow, 0)
        for r in range(RPT // BATCH):
            pltpu.sync_copy(ones_v.at[pl.ds(0, BATCH)],
                            acc_sh.at[pl.ds(s * RPT + r * BATCH, BATCH)])
        pltpu.sync_copy(ones_hbm, ones_v)
        plsc.subcore_barrier()

        def body(i, _):
            pltpu.sync_copy(
                idx_hbm.at[1, pl.ds((ibase + i * NBC_D) * BATCH,
                                    NBC_D * BATCH)],
                idx_v.at[1])
            pltpu.sync_copy(ones_v, acc_sh.at[idx_v.at[1]], add=True)
            return 0

        lax.fori_loop(0, NB // NBC_D, body, 0)
        plsc.subcore_barrier()
        pltpu.sync_copy(acc_sh.at[pl.ds(s * RPT, RPT)],
                        out_hbm.at[c, pl.ds(s * RPT, RPT)])

    return deg


def _zero_init(rows0_v, acc_sh, s):
    def zrow(i, _):
        for j in range(DH // 16):
            rows0_v[i, pl.ds(j * 16, 16)] = jnp.zeros((16,), jnp.float32)
        return 0
    lax.fori_loop(0, BATCH, zrow, 0)
    for r in range(RPT // BATCH):
        pltpu.sync_copy(rows0_v.at[pl.ds(0, BATCH)],
                        acc_sh.at[pl.ds(s * RPT + r * BATCH, BATCH)])


def _copy_idx(idx_hbm, idx_v, mb):
    for p in (0, 1):
        pltpu.sync_copy(idx_hbm.at[p, pl.ds(mb * EPT, EPT)], idx_v.at[p])


def _edge_loop(idx_hbm, table_sh, acc_sh, bufs, mb_base, nmb):
    for b in (0, 1):
        idx_v, rows_v, sem = bufs[b]
        _copy_idx(idx_hbm, idx_v, mb_base + b)
        pltpu.async_copy(table_sh.at[idx_v.at[0]], rows_v, sem)

    def body(k, _):
        for b in (0, 1):
            i = 2 * k + b
            idx_v, rows_v, sem = bufs[b]
            pltpu.make_async_copy(
                table_sh.at[idx_v.at[0]], rows_v, sem).wait()
            pltpu.sync_copy(rows_v, acc_sh.at[idx_v.at[1]], add=True)

            @pl.when(i + 2 < nmb)
            def _():
                _copy_idx(idx_hbm, idx_v, mb_base + i + 2)
                pltpu.async_copy(table_sh.at[idx_v.at[0]], rows_v, sem)
        return 0

    lax.fori_loop(0, nmb // 2, body, 0)


_AGG_SCRATCH = [
    pltpu.VMEM((2, EPT), jnp.int32),
    pltpu.VMEM((2, EPT), jnp.int32),
    pltpu.VMEM((EPT, DH), jnp.float32),
    pltpu.VMEM((EPT, DH), jnp.float32),
    pltpu.VMEM_SHARED((N_PAD, DH), jnp.float32),
    pltpu.VMEM_SHARED((N_PAD, DH), jnp.float32),
    pltpu.SemaphoreType.DMA,
    pltpu.SemaphoreType.DMA,
]


def _agg_fsplit_kernel():
    mesh = plsc.VectorSubcoreMesh(core_axis_name="c", subcore_axis_name="s")

    @functools.partial(
        pl.kernel,
        out_type=jax.ShapeDtypeStruct((NC, N_PAD, DH), jnp.float32),
        mesh=mesh,
        scratch_types=_AGG_SCRATCH,
        compiler_params=pltpu.CompilerParams(use_tc_tiling_on_sc=False),
    )
    def agg(table_hbm, idx_hbm, out_hbm,
            idx0_v, idx1_v, rows0_v, rows1_v, table_sh, acc_sh, sem0, sem1):
        c = lax.axis_index("c")
        s = lax.axis_index("s")
        bufs = ((idx0_v, rows0_v, sem0), (idx1_v, rows1_v, sem1))
        pltpu.sync_copy(table_hbm.at[c, pl.ds(s * SRT, SRT)],
                        table_sh.at[pl.ds(s * SRT, SRT)])
        _zero_init(rows0_v, acc_sh, s)
        plsc.subcore_barrier()
        _edge_loop(idx_hbm, table_sh, acc_sh, bufs,
                   s * (E_PAD // NS // EPT), E_PAD // NS // EPT)
        plsc.subcore_barrier()
        pltpu.sync_copy(acc_sh.at[pl.ds(s * RPT, RPT)],
                        out_hbm.at[c, pl.ds(s * RPT, RPT)])

    return agg


def _agg_esplit_kernel():
    mesh = plsc.VectorSubcoreMesh(core_axis_name="c", subcore_axis_name="s")

    @functools.partial(
        pl.kernel,
        out_type=jax.ShapeDtypeStruct((NC, N_PAD, DH), jnp.float32),
        mesh=mesh,
        scratch_types=_AGG_SCRATCH,
        compiler_params=pltpu.CompilerParams(use_tc_tiling_on_sc=False),
    )
    def agg(table_hbm, idx_hbm, out_hbm,
            idx0_v, idx1_v, rows0_v, rows1_v, table_sh, acc_sh, sem0, sem1):
        c = lax.axis_index("c")
        s = lax.axis_index("s")
        wid = s * NC + c
        bufs = ((idx0_v, rows0_v, sem0), (idx1_v, rows1_v, sem1))
        pltpu.sync_copy(table_hbm.at[pl.ds(s * SRT, SRT)],
                        table_sh.at[pl.ds(s * SRT, SRT)])
        _zero_init(rows0_v, acc_sh, s)
        plsc.subcore_barrier()
        _edge_loop(idx_hbm, table_sh, acc_sh, bufs,
                   wid * (EP // EPT), EP // EPT)
        plsc.subcore_barrier()
        pltpu.sync_copy(acc_sh.at[pl.ds(s * RPT, RPT)],
                        out_hbm.at[c, pl.ds(s * RPT, RPT)])

    return agg


_ROWS_BLK = 1000
_GRID = N // _ROWS_BLK


def _dis_from(degp_blk):
    deg = degp_blk[0, :, :1] + degp_blk[1, :, :1] + 1.0
    return lax.rsqrt(deg)


def _tc1a_body(x_ref, w1_ref, h_ref):
    h_ref[...] = jnp.dot(x_ref[...], w1_ref[...],
                         preferred_element_type=jnp.float32)


def _tc1b_body(degp_ref, h_ref, g1_ref):
    dis = _dis_from(degp_ref[...])
    h = h_ref[...]
    g1_ref[0] = dis * h[:, :DH]
    g1_ref[1] = dis * h[:, DH:]


def _tc2_body(degp_ref, r1_ref, g1_ref, b1_ref, w2_ref, g2_ref):
    dis = _dis_from(degp_ref[...])
    a_lo = dis * (r1_ref[0] + g1_ref[0]) + b1_ref[:, :DH]
    a_hi = dis * (r1_ref[1] + g1_ref[1]) + b1_ref[:, DH:]
    x2 = jnp.concatenate([jnp.maximum(a_lo, 0.0), jnp.maximum(a_hi, 0.0)],
                         axis=1)
    g2_ref[...] = dis * jnp.dot(x2, w2_ref[...],
                                preferred_element_type=jnp.float32)


def _tc3_body(degp_ref, r2_ref, g2_ref, b2_ref, out_ref):
    dis = _dis_from(degp_ref[...])
    out_ref[...] = dis * (r2_ref[0] + r2_ref[1] + g2_ref[...]) + b2_ref[...]


def _blk_parts(d):
    return pl.BlockSpec((NC, _ROWS_BLK, d), lambda i: (0, i, 0))


def _blk_rows(d):
    return pl.BlockSpec((_ROWS_BLK, d), lambda i: (i, 0))


def _blk_full(shape):
    return pl.BlockSpec(shape, lambda i: tuple(0 for _ in shape))


def kernel(x, edge_index, W1, b1, W2, b2):
    pad = E_PAD - E
    pad_block = jnp.concatenate(
        [jnp.zeros((1, pad), jnp.int32), jnp.full((1, pad), N, jnp.int32)])
    idx = jnp.concatenate([edge_index, pad_block], axis=1)
    ones16 = jnp.ones((8 * BATCH, 16), jnp.float32)

    degp = _deg_kernel()(idx, ones16)

    h1 = pl.pallas_call(
        _tc1a_body,
        grid=(_GRID,),
        in_specs=[_blk_rows(D_IN), _blk_full((D_IN, D_HID))],
        out_specs=_blk_rows(D_HID),
        out_shape=jax.ShapeDtypeStruct((N, D_HID), jnp.float32),
    )(x, W1)

    g1 = pl.pallas_call(
        _tc1b_body,
        grid=(_GRID,),
        in_specs=[_blk_parts(16), _blk_rows(D_HID)],
        out_specs=_blk_parts(DH),
        out_shape=jax.ShapeDtypeStruct((NC, N, DH), jnp.float32),
    )(degp, h1)

    r1 = _agg_fsplit_kernel()(g1, idx)

    g2 = pl.pallas_call(
        _tc2_body,
        grid=(_GRID,),
        in_specs=[_blk_parts(16), _blk_parts(DH), _blk_parts(DH),
                  _blk_full((1, D_HID)), _blk_full((D_HID, D_OUT))],
        out_specs=_blk_rows(D_OUT),
        out_shape=jax.ShapeDtypeStruct((N, D_OUT), jnp.float32),
    )(degp, r1, g1, b1.reshape(1, D_HID), W2)

    r2 = _agg_esplit_kernel()(g2, idx)

    out = pl.pallas_call(
        _tc3_body,
        grid=(_GRID,),
        in_specs=[_blk_parts(16), _blk_parts(D_OUT), _blk_rows(D_OUT),
                  _blk_full((1, D_OUT))],
        out_specs=_blk_rows(D_OUT),
        out_shape=jax.ShapeDtypeStruct((N, D_OUT), jnp.float32),
    )(degp, r2, g2, b2.reshape(1, D_OUT))

    return out


# speedup vs baseline: 1.1432x; 1.0174x over previous
import functools

import jax
import jax.numpy as jnp
from jax import lax
from jax.experimental import pallas as pl
from jax.experimental.pallas import tpu as pltpu
from jax.experimental.pallas import tpu_sc as plsc

N = 10000
E = 320000
D_IN = 128
D_HID = 128
D_OUT = 64
DH = 64

NC = 2
NS = 16
NW = NC * NS

BATCH = 128
EPT = 320
NB = 80
EP = NB * BATCH
E_PAD = EP * NW
NBT = E_PAD // BATCH
NB_F = NBT // NS
N_PAD = 10240
RPT = N_PAD // NS
SRT = N // NS


def _deg_kernel():
    mesh = plsc.VectorSubcoreMesh(core_axis_name="c", subcore_axis_name="s")

    NBC_D = 8

    @functools.partial(
        pl.kernel,
        out_type=jax.ShapeDtypeStruct((NC, N_PAD, 16), jnp.float32),
        mesh=mesh,
        scratch_types=[
            pltpu.VMEM((2, NBC_D * BATCH), jnp.int32),
            pltpu.VMEM((NBC_D * BATCH, 16), jnp.float32),
            pltpu.VMEM_SHARED((N_PAD, 16), jnp.float32),
        ],
        compiler_params=pltpu.CompilerParams(use_tc_tiling_on_sc=False),
    )
    def deg(idx_hbm, ones_hbm, out_hbm, idx_v, ones_v, acc_sh):
        c = lax.axis_index("c")
        s = lax.axis_index("s")
        wid = s * NC + c
        ibase = wid * NB
        def zrow(i, _):
            ones_v[i, :] = jnp.zeros((16,), jnp.float32)
            return 0
        lax.fori_loop(0, BATCH, zrow, 0)
        for r in range(RPT // BATCH):
            pltpu.sync_copy(ones_v.at[pl.ds(0, BATCH)],
                            acc_sh.at[pl.ds(s * RPT + r * BATCH, BATCH)])
        pltpu.sync_copy(ones_hbm, ones_v)
        plsc.subcore_barrier()

        def body(i, _):
            pltpu.sync_copy(
                idx_hbm.at[1, pl.ds((ibase + i * NBC_D) * BATCH,
                                    NBC_D * BATCH)],
                idx_v.at[1])
            pltpu.sync_copy(ones_v, acc_sh.at[idx_v.at[1]], add=True)
            return 0

        lax.fori_loop(0, NB // NBC_D, body, 0)
        plsc.subcore_barrier()
        pltpu.sync_copy(acc_sh.at[pl.ds(s * RPT, RPT)],
                        out_hbm.at[c, pl.ds(s * RPT, RPT)])

    return deg


def _zero_init(rows0_v, acc_sh, s):
    def zrow(i, _):
        for j in range(DH // 16):
            rows0_v[i, pl.ds(j * 16, 16)] = jnp.zeros((16,), jnp.float32)
        return 0
    lax.fori_loop(0, BATCH, zrow, 0)
    for r in range(RPT // BATCH):
        pltpu.sync_copy(rows0_v.at[pl.ds(0, BATCH)],
                        acc_sh.at[pl.ds(s * RPT + r * BATCH, BATCH)])


def _copy_idx(idx_hbm, idx_v, mb):
    for p in (0, 1):
        pltpu.sync_copy(idx_hbm.at[p, pl.ds(mb * EPT, EPT)], idx_v.at[p])


def _edge_loop(idx_hbm, table_sh, acc_sh, bufs, mb_base, nmb):
    for b in (0, 1):
        idx_v, rows_v, sem = bufs[b]
        _copy_idx(idx_hbm, idx_v, mb_base + b)
        pltpu.async_copy(table_sh.at[idx_v.at[0]], rows_v, sem)

    def body(k, _):
        for b in (0, 1):
            i = 2 * k + b
            idx_v, rows_v, sem = bufs[b]
            pltpu.make_async_copy(
                table_sh.at[idx_v.at[0]], rows_v, sem).wait()
            pltpu.sync_copy(rows_v, acc_sh.at[idx_v.at[1]], add=True)

            @pl.when(i + 2 < nmb)
            def _():
                _copy_idx(idx_hbm, idx_v, mb_base + i + 2)
                pltpu.async_copy(table_sh.at[idx_v.at[0]], rows_v, sem)
        return 0

    lax.fori_loop(0, nmb // 2, body, 0)


_AGG_SCRATCH = [
    pltpu.VMEM((2, EPT), jnp.int32),
    pltpu.VMEM((2, EPT), jnp.int32),
    pltpu.VMEM((EPT, DH), jnp.float32),
    pltpu.VMEM((EPT, DH), jnp.float32),
    pltpu.VMEM_SHARED((N_PAD, DH), jnp.float32),
    pltpu.VMEM_SHARED((N_PAD, DH), jnp.float32),
    pltpu.SemaphoreType.DMA,
    pltpu.SemaphoreType.DMA,
]


def _agg_fsplit_kernel():
    mesh = plsc.VectorSubcoreMesh(core_axis_name="c", subcore_axis_name="s")

    @functools.partial(
        pl.kernel,
        out_type=jax.ShapeDtypeStruct((NC, N_PAD, DH), jnp.float32),
        mesh=mesh,
        scratch_types=_AGG_SCRATCH,
        compiler_params=pltpu.CompilerParams(use_tc_tiling_on_sc=False),
    )
    def agg(table_hbm, idx_hbm, out_hbm,
            idx0_v, idx1_v, rows0_v, rows1_v, table_sh, acc_sh, sem0, sem1):
        c = lax.axis_index("c")
        s = lax.axis_index("s")
        bufs = ((idx0_v, rows0_v, sem0), (idx1_v, rows1_v, sem1))
        pltpu.sync_copy(table_hbm.at[c, pl.ds(s * SRT, SRT)],
                        table_sh.at[pl.ds(s * SRT, SRT)])
        _zero_init(rows0_v, acc_sh, s)
        plsc.subcore_barrier()
        _edge_loop(idx_hbm, table_sh, acc_sh, bufs,
                   s * (E_PAD // NS // EPT), E_PAD // NS // EPT)
        plsc.subcore_barrier()
        pltpu.sync_copy(acc_sh.at[pl.ds(s * RPT, RPT)],
                        out_hbm.at[c, pl.ds(s * RPT, RPT)])

    return agg


def _agg_esplit_kernel():
    mesh = plsc.VectorSubcoreMesh(core_axis_name="c", subcore_axis_name="s")

    @functools.partial(
        pl.kernel,
        out_type=jax.ShapeDtypeStruct((NC, N_PAD, DH), jnp.float32),
        mesh=mesh,
        scratch_types=_AGG_SCRATCH,
        compiler_params=pltpu.CompilerParams(use_tc_tiling_on_sc=False),
    )
    def agg(table_hbm, idx_hbm, out_hbm,
            idx0_v, idx1_v, rows0_v, rows1_v, table_sh, acc_sh, sem0, sem1):
        c = lax.axis_index("c")
        s = lax.axis_index("s")
        wid = s * NC + c
        bufs = ((idx0_v, rows0_v, sem0), (idx1_v, rows1_v, sem1))
        pltpu.sync_copy(table_hbm.at[pl.ds(s * SRT, SRT)],
                        table_sh.at[pl.ds(s * SRT, SRT)])
        _zero_init(rows0_v, acc_sh, s)
        plsc.subcore_barrier()
        _edge_loop(idx_hbm, table_sh, acc_sh, bufs,
                   wid * (EP // EPT), EP // EPT)
        plsc.subcore_barrier()
        pltpu.sync_copy(acc_sh.at[pl.ds(s * RPT, RPT)],
                        out_hbm.at[c, pl.ds(s * RPT, RPT)])

    return agg


_ROWS_BLK = 2000
_GRID = N // _ROWS_BLK


def _dis_from(degp_blk):
    deg = degp_blk[0, :, :1] + degp_blk[1, :, :1] + 1.0
    return lax.rsqrt(deg)


def _tc1a_body(x_ref, w1_ref, h_ref):
    h_ref[...] = jnp.dot(x_ref[...], w1_ref[...],
                         preferred_element_type=jnp.float32)


def _tc1b_body(degp_ref, h_ref, g1_ref):
    dis = _dis_from(degp_ref[...])
    h = h_ref[...]
    g1_ref[0] = dis * h[:, :DH]
    g1_ref[1] = dis * h[:, DH:]


def _tc2_body(degp_ref, r1_ref, g1_ref, b1_ref, w2_ref, g2_ref):
    dis = _dis_from(degp_ref[...])
    a_lo = dis * (r1_ref[0] + g1_ref[0]) + b1_ref[:, :DH]
    a_hi = dis * (r1_ref[1] + g1_ref[1]) + b1_ref[:, DH:]
    x2 = jnp.concatenate([jnp.maximum(a_lo, 0.0), jnp.maximum(a_hi, 0.0)],
                         axis=1)
    g2_ref[...] = dis * jnp.dot(x2, w2_ref[...],
                                preferred_element_type=jnp.float32)


def _tc3_body(degp_ref, r2_ref, g2_ref, b2_ref, out_ref):
    dis = _dis_from(degp_ref[...])
    out_ref[...] = dis * (r2_ref[0] + r2_ref[1] + g2_ref[...]) + b2_ref[...]


def _blk_parts(d):
    return pl.BlockSpec((NC, _ROWS_BLK, d), lambda i: (0, i, 0))


def _blk_rows(d):
    return pl.BlockSpec((_ROWS_BLK, d), lambda i: (i, 0))


def _blk_full(shape):
    return pl.BlockSpec(shape, lambda i: tuple(0 for _ in shape))


def kernel(x, edge_index, W1, b1, W2, b2):
    pad = E_PAD - E
    pad_block = jnp.concatenate(
        [jnp.zeros((1, pad), jnp.int32), jnp.full((1, pad), N, jnp.int32)])
    idx = jnp.concatenate([edge_index, pad_block], axis=1)
    ones16 = jnp.ones((8 * BATCH, 16), jnp.float32)

    degp = _deg_kernel()(idx, ones16)

    h1 = pl.pallas_call(
        _tc1a_body,
        grid=(_GRID,),
        in_specs=[_blk_rows(D_IN), _blk_full((D_IN, D_HID))],
        out_specs=_blk_rows(D_HID),
        out_shape=jax.ShapeDtypeStruct((N, D_HID), jnp.float32),
    )(x, W1)

    g1 = pl.pallas_call(
        _tc1b_body,
        grid=(_GRID,),
        in_specs=[_blk_parts(16), _blk_rows(D_HID)],
        out_specs=_blk_parts(DH),
        out_shape=jax.ShapeDtypeStruct((NC, N, DH), jnp.float32),
    )(degp, h1)

    r1 = _agg_fsplit_kernel()(g1, idx)

    g2 = pl.pallas_call(
        _tc2_body,
        grid=(_GRID,),
        in_specs=[_blk_parts(16), _blk_parts(DH), _blk_parts(DH),
                  _blk_full((1, D_HID)), _blk_full((D_HID, D_OUT))],
        out_specs=_blk_rows(D_OUT),
        out_shape=jax.ShapeDtypeStruct((N, D_OUT), jnp.float32),
    )(degp, r1, g1, b1.reshape(1, D_HID), W2)

    r2 = _agg_esplit_kernel()(g2, idx)

    out = pl.pallas_call(
        _tc3_body,
        grid=(_GRID,),
        in_specs=[_blk_parts(16), _blk_parts(D_OUT), _blk_rows(D_OUT),
                  _blk_full((1, D_OUT))],
        out_specs=_blk_rows(D_OUT),
        out_shape=jax.ShapeDtypeStruct((N, D_OUT), jnp.float32),
    )(degp, r2, g2, b2.reshape(1, D_OUT))

    return out
